# baseline (speedup 1.0000x reference)
import functools

import jax
import jax.numpy as jnp
from jax import lax
from jax.experimental import pallas as pl
from jax.experimental.pallas import tpu as pltpu
from jax.experimental.pallas import tpu_sc as plsc

NC = 2
NS = 16
NW = NC * NS
LANES = 16


def _sc_scores(ctx_idx, tgt_idx, neg_idx, emb_in, emb_out, B, CTX, NEG, D):
    bpw = B // NW
    CB = 16
    nch = bpw // CB
    nseg = D // LANES
    inv_ctx = jnp.float32(1.0 / CTX)

    mesh = plsc.VectorSubcoreMesh(core_axis_name="c", subcore_axis_name="s")

    @functools.partial(
        pl.kernel,
        mesh=mesh,
        compiler_params=pltpu.CompilerParams(
            needs_layout_passes=False, use_tc_tiling_on_sc=False),
        out_type=jax.ShapeDtypeStruct((B, 32), jnp.float32),
        scratch_types=[
            pltpu.VMEM((2, CB, CTX), jnp.int32),
            pltpu.VMEM((2, CB, NEG), jnp.int32),
            pltpu.VMEM((2, CB), jnp.int32),
            pltpu.VMEM((2, CB * CTX, D), jnp.float32),
            pltpu.VMEM((2, CB * NEG, D), jnp.float32),
            pltpu.VMEM((2, CB, D), jnp.float32),
            pltpu.VMEM((CB, D), jnp.float32),
            pltpu.VMEM((2, CB, 32), jnp.float32),
            pltpu.SemaphoreType.DMA,
            pltpu.SemaphoreType.DMA,
            pltpu.SemaphoreType.DMA,
            pltpu.SemaphoreType.DMA,
            pltpu.SemaphoreType.DMA,
            pltpu.SemaphoreType.DMA,
        ],
    )
    def sc_kernel(ctx_i_hbm, tgt_i_hbm, neg_i_hbm, ein_hbm, eout_hbm, out_hbm,
                  ctxi_v, negi_v, tgti_v, ctxr_v, negr_v, tgtr_v, ctxv_v,
                  sc_v, sem0, sem1, semi0, semi1, semo0, semo1):
        wid = lax.axis_index("s") * NC + lax.axis_index("c")
        sems = (sem0, sem1)
        semis = (semi0, semi1)
        semos = (semo0, semo1)

        def idx_load(c, p):
            b0 = wid * bpw + c * CB
            pltpu.async_copy(ctx_i_hbm.at[pl.ds(b0, CB)], ctxi_v.at[p],
                             semis[p])
            pltpu.async_copy(neg_i_hbm.at[pl.ds(b0, CB)], negi_v.at[p],
                             semis[p])
            pltpu.async_copy(tgt_i_hbm.at[pl.ds(b0, CB)], tgti_v.at[p],
                             semis[p])

        def idx_wait(p):
            b0 = wid * bpw
            pltpu.make_async_copy(ctx_i_hbm.at[pl.ds(b0, CB)], ctxi_v.at[p],
                                  semis[p]).wait()
            pltpu.make_async_copy(neg_i_hbm.at[pl.ds(b0, CB)], negi_v.at[p],
                                  semis[p]).wait()
            pltpu.make_async_copy(tgt_i_hbm.at[pl.ds(b0, CB)], tgti_v.at[p],
                                  semis[p]).wait()

        def fire(c, p):
            sem = sems[p]
            for b in range(CB):
                pltpu.async_copy(ein_hbm.at[ctxi_v.at[p, b]],
                                 ctxr_v.at[p, pl.ds(b * CTX, CTX)], sem)
            for b in range(CB):
                pltpu.async_copy(eout_hbm.at[negi_v.at[p, b]],
                                 negr_v.at[p, pl.ds(b * NEG, NEG)], sem)
            pltpu.async_copy(eout_hbm.at[tgti_v.at[p]], tgtr_v.at[p], sem)

        def drain(p):
            sem = sems[p]
            for b in range(CB):
                pltpu.make_async_copy(ein_hbm.at[ctxi_v.at[p, b]],
                                      ctxr_v.at[p, pl.ds(b * CTX, CTX)],
                                      sem).wait()
            for b in range(CB):
                pltpu.make_async_copy(eout_hbm.at[negi_v.at[p, b]],
                                      negr_v.at[p, pl.ds(b * NEG, NEG)],
                                      sem).wait()
            pltpu.make_async_copy(eout_hbm.at[tgti_v.at[p]], tgtr_v.at[p],
                                  sem).wait()

        def score_out(c, p):
            pltpu.async_copy(sc_v.at[p],
                             out_hbm.at[pl.ds(wid * bpw + c * CB, CB)],
                             semos[p])

        def score_wait(p):
            pltpu.make_async_copy(sc_v.at[p],
                                  out_hbm.at[pl.ds(wid * bpw, CB)],
                                  semos[p]).wait()

        lane = lax.iota(jnp.int32, LANES)
        lane_neg = lane * NEG
        zero = jnp.zeros((LANES,), jnp.float32)

        def compute(c, p):
            def bbody(b, carry):
                base = b * CTX
                for s in range(nseg):
                    v = [ctxr_v[p, base + j, pl.ds(s * LANES, LANES)]
                         for j in range(CTX)]
                    while len(v) > 1:
                        v = [v[i] + v[i + 1] for i in range(0, len(v) - 1, 2)] \
                            + ([v[-1]] if len(v) % 2 else [])
                    ctxv_v[b, pl.ds(s * LANES, LANES)] = v[0] * inv_ctx
                return carry

            lax.fori_loop(0, CB, bbody, 0, unroll=2)

            def dbody(d, carry):
                pos = carry[0]
                negs = carry[1:]
                dcol = jnp.broadcast_to(d, (LANES,))
                acc = plsc.load_gather(ctxv_v, [lane, dcol])
                tcol = plsc.load_gather(tgtr_v.at[p], [lane, dcol])
                pos = pos + acc * tcol
                negs = list(negs)
                for kb in range(0, NEG, 4):
                    kk = range(kb, min(kb + 4, NEG))
                    gs = [plsc.load_gather(negr_v.at[p], [lane_neg + k, dcol])
                          for k in kk]
                    for k, g in zip(kk, gs):
                        negs[k] = negs[k] + acc * g
                return [pos] + negs

            res = lax.fori_loop(0, D, dbody, [zero] * (NEG + 1), unroll=2)
            plsc.store_scatter(sc_v.at[p],
                               [lane, jnp.broadcast_to(0, (LANES,))], -res[0])
            for k in range(NEG):
                plsc.store_scatter(sc_v.at[p],
                                   [lane, jnp.broadcast_to(k + 1, (LANES,))],
                                   res[k + 1])

        for q in range(2):
            for z in range(CB):
                sc_v[q, z, pl.ds(16, 16)] = jnp.zeros((LANES,), jnp.float32)

        idx_load(0, 0)
        idx_wait(0)
        fire(0, 0)
        idx_load(1, 1)

        def pair(i, carry):
            for pp in range(2):
                c = i * 2 + pp
                drain(pp)

                @pl.when(c + 2 < nch)
                def _():
                    idx_load(c + 2, pp)

                @pl.when(c + 1 < nch)
                def _():
                    idx_wait(1 - pp)
                    fire(c + 1, 1 - pp)

                @pl.when(c >= 2)
                def _():
                    score_wait(pp)

                compute(c, pp)
                score_out(c, pp)
            return carry

        lax.fori_loop(0, nch // 2, pair, 0)
        score_wait(0)
        score_wait(1)

    return sc_kernel(ctx_idx, tgt_idx, neg_idx, emb_in, emb_out)


def _tc_loss(scores, B, NEG):
    RB = 2048
    grid = B // RB

    def body(s_ref, o_ref):
        i = pl.program_id(0)
        x = s_ref[...]
        col = lax.broadcasted_iota(jnp.int32, x.shape, 1)
        sp = jnp.maximum(x, 0.0) + jnp.log1p(jnp.exp(-jnp.abs(x)))
        sp = jnp.where(col < NEG + 1, sp, 0.0)
        part = jnp.sum(sp)

        @pl.when(i == 0)
        def _():
            o_ref[0, 0] = 0.0

        o_ref[0, 0] += part

        @pl.when(i == grid - 1)
        def _():
            o_ref[0, 0] = o_ref[0, 0] * jnp.float32(1.0 / B)

    return pl.pallas_call(
        body,
        grid=(grid,),
        in_specs=[pl.BlockSpec((RB, 32), lambda i: (i, 0))],
        out_specs=pl.BlockSpec(memory_space=pltpu.SMEM),
        out_shape=jax.ShapeDtypeStruct((1, 1), jnp.float32),
    )(scores)


def kernel(context_words, target_words, negative_samples, emb_in, emb_out):
    B, CTX = context_words.shape
    NEG = negative_samples.shape[1]
    D = emb_in.shape[1]

    ctx_idx = context_words.astype(jnp.int32)
    neg_idx = negative_samples.astype(jnp.int32)
    tgt_idx = target_words.astype(jnp.int32)

    scores = _sc_scores(ctx_idx, tgt_idx, neg_idx, emb_in, emb_out,
                        B, CTX, NEG, D)
    loss = _tc_loss(scores, B, NEG)
    return loss[0, 0]


# speedup vs baseline: 1.0142x; 1.0142x over previous
import functools

import jax
import jax.numpy as jnp
from jax import lax
from jax.experimental import pallas as pl
from jax.experimental.pallas import tpu as pltpu
from jax.experimental.pallas import tpu_sc as plsc

NC = 2
NS = 16
NW = NC * NS
LANES = 16


def _sc_scores(ctx_idx, tgt_idx, neg_idx, emb_in, emb_out, B, CTX, NEG, D):
    bpw = B // NW
    CB = 16
    nch = bpw // CB
    nseg = D // LANES
    inv_ctx = jnp.float32(1.0 / CTX)

    mesh = plsc.VectorSubcoreMesh(core_axis_name="c", subcore_axis_name="s")

    @functools.partial(
        pl.kernel,
        mesh=mesh,
        compiler_params=pltpu.CompilerParams(
            needs_layout_passes=False, use_tc_tiling_on_sc=False),
        out_type=jax.ShapeDtypeStruct((B, 32), jnp.float32),
        scratch_types=[
            pltpu.VMEM((2, CB, CTX), jnp.int32),
            pltpu.VMEM((2, CB, NEG), jnp.int32),
            pltpu.VMEM((2, CB), jnp.int32),
            pltpu.VMEM((2, CB * CTX, D), jnp.float32),
            pltpu.VMEM((2, CB * NEG, D), jnp.float32),
            pltpu.VMEM((2, CB, D), jnp.float32),
            pltpu.VMEM((CB, D), jnp.float32),
            pltpu.VMEM((2, CB, 32), jnp.float32),
            pltpu.SemaphoreType.DMA,
            pltpu.SemaphoreType.DMA,
            pltpu.SemaphoreType.DMA,
            pltpu.SemaphoreType.DMA,
            pltpu.SemaphoreType.DMA,
            pltpu.SemaphoreType.DMA,
        ],
    )
    def sc_kernel(ctx_i_hbm, tgt_i_hbm, neg_i_hbm, ein_hbm, eout_hbm, out_hbm,
                  ctxi_v, negi_v, tgti_v, ctxr_v, negr_v, tgtr_v, ctxv_v,
                  sc_v, sem0, sem1, semi0, semi1, semo0, semo1):
        wid = lax.axis_index("s") * NC + lax.axis_index("c")
        sems = (sem0, sem1)
        semis = (semi0, semi1)
        semos = (semo0, semo1)

        def idx_load(c, p):
            b0 = wid * bpw + c * CB
            pltpu.async_copy(ctx_i_hbm.at[pl.ds(b0, CB)], ctxi_v.at[p],
                             semis[p])
            pltpu.async_copy(neg_i_hbm.at[pl.ds(b0, CB)], negi_v.at[p],
                             semis[p])
            pltpu.async_copy(tgt_i_hbm.at[pl.ds(b0, CB)], tgti_v.at[p],
                             semis[p])

        def idx_wait(p):
            b0 = wid * bpw
            pltpu.make_async_copy(ctx_i_hbm.at[pl.ds(b0, CB)], ctxi_v.at[p],
                                  semis[p]).wait()
            pltpu.make_async_copy(neg_i_hbm.at[pl.ds(b0, CB)], negi_v.at[p],
                                  semis[p]).wait()
            pltpu.make_async_copy(tgt_i_hbm.at[pl.ds(b0, CB)], tgti_v.at[p],
                                  semis[p]).wait()

        def fire(c, p):
            sem = sems[p]
            for b in range(CB):
                pltpu.async_copy(ein_hbm.at[ctxi_v.at[p, b]],
                                 ctxr_v.at[p, pl.ds(b * CTX, CTX)], sem)
            for b in range(CB):
                pltpu.async_copy(eout_hbm.at[negi_v.at[p, b]],
                                 negr_v.at[p, pl.ds(b * NEG, NEG)], sem)
            pltpu.async_copy(eout_hbm.at[tgti_v.at[p]], tgtr_v.at[p], sem)

        def drain(p):
            sem = sems[p]
            for b in range(CB):
                pltpu.make_async_copy(ein_hbm.at[ctxi_v.at[p, b]],
                                      ctxr_v.at[p, pl.ds(b * CTX, CTX)],
                                      sem).wait()
            for b in range(CB):
                pltpu.make_async_copy(eout_hbm.at[negi_v.at[p, b]],
                                      negr_v.at[p, pl.ds(b * NEG, NEG)],
                                      sem).wait()
            pltpu.make_async_copy(eout_hbm.at[tgti_v.at[p]], tgtr_v.at[p],
                                  sem).wait()

        def score_out(c, p):
            pltpu.async_copy(sc_v.at[p],
                             out_hbm.at[pl.ds(wid * bpw + c * CB, CB)],
                             semos[p])

        def score_wait(p):
            pltpu.make_async_copy(sc_v.at[p],
                                  out_hbm.at[pl.ds(wid * bpw, CB)],
                                  semos[p]).wait()

        lane = lax.iota(jnp.int32, LANES)
        lane_neg = lane * NEG
        zero = jnp.zeros((LANES,), jnp.float32)

        def compute(c, p):
            def bbody(b, carry):
                base = b * CTX
                for s in range(nseg):
                    v = [ctxr_v[p, base + j, pl.ds(s * LANES, LANES)]
                         for j in range(CTX)]
                    while len(v) > 1:
                        v = [v[i] + v[i + 1] for i in range(0, len(v) - 1, 2)] \
                            + ([v[-1]] if len(v) % 2 else [])
                    ctxv_v[b, pl.ds(s * LANES, LANES)] = v[0] * inv_ctx
                return carry

            lax.fori_loop(0, CB, bbody, 0, unroll=2)

            def dbody(d, carry):
                pos = carry[0]
                negs = carry[1:]
                dcol = jnp.broadcast_to(d, (LANES,))
                acc = plsc.load_gather(ctxv_v, [lane, dcol])
                tcol = plsc.load_gather(tgtr_v.at[p], [lane, dcol])
                gs = [plsc.load_gather(negr_v.at[p], [lane_neg + k, dcol])
                      for k in range(NEG)]
                pos = pos + acc * tcol
                negs = [n + acc * g for n, g in zip(negs, gs)]
                return [pos] + negs

            res = lax.fori_loop(0, D, dbody, [zero] * (NEG + 1), unroll=2)
            plsc.store_scatter(sc_v.at[p],
                               [lane, jnp.broadcast_to(0, (LANES,))], -res[0])
            for k in range(NEG):
                plsc.store_scatter(sc_v.at[p],
                                   [lane, jnp.broadcast_to(k + 1, (LANES,))],
                                   res[k + 1])

        for q in range(2):
            for z in range(CB):
                sc_v[q, z, pl.ds(16, 16)] = jnp.zeros((LANES,), jnp.float32)

        idx_load(0, 0)
        idx_wait(0)
        fire(0, 0)
        idx_load(1, 1)

        def pair(i, carry):
            for pp in range(2):
                c = i * 2 + pp
                drain(pp)

                @pl.when(c + 2 < nch)
                def _():
                    idx_load(c + 2, pp)

                @pl.when(c + 1 < nch)
                def _():
                    idx_wait(1 - pp)
                    fire(c + 1, 1 - pp)

                @pl.when(c >= 2)
                def _():
                    score_wait(pp)

                compute(c, pp)
                score_out(c, pp)
            return carry

        lax.fori_loop(0, nch // 2, pair, 0)
        score_wait(0)
        score_wait(1)

    return sc_kernel(ctx_idx, tgt_idx, neg_idx, emb_in, emb_out)


def _tc_loss(scores, B, NEG):
    RB = 2048
    grid = B // RB

    def body(s_ref, o_ref):
        i = pl.program_id(0)
        x = s_ref[...]
        col = lax.broadcasted_iota(jnp.int32, x.shape, 1)
        sp = jnp.maximum(x, 0.0) + jnp.log1p(jnp.exp(-jnp.abs(x)))
        sp = jnp.where(col < NEG + 1, sp, 0.0)
        part = jnp.sum(sp)

        @pl.when(i == 0)
        def _():
            o_ref[0, 0] = 0.0

        o_ref[0, 0] += part

        @pl.when(i == grid - 1)
        def _():
            o_ref[0, 0] = o_ref[0, 0] * jnp.float32(1.0 / B)

    return pl.pallas_call(
        body,
        grid=(grid,),
        in_specs=[pl.BlockSpec((RB, 32), lambda i: (i, 0))],
        out_specs=pl.BlockSpec(memory_space=pltpu.SMEM),
        out_shape=jax.ShapeDtypeStruct((1, 1), jnp.float32),
    )(scores)


def kernel(context_words, target_words, negative_samples, emb_in, emb_out):
    B, CTX = context_words.shape
    NEG = negative_samples.shape[1]
    D = emb_in.shape[1]

    ctx_idx = context_words.astype(jnp.int32)
    neg_idx = negative_samples.astype(jnp.int32)
    tgt_idx = target_words.astype(jnp.int32)

    scores = _sc_scores(ctx_idx, tgt_idx, neg_idx, emb_in, emb_out,
                        B, CTX, NEG, D)
    loss = _tc_loss(scores, B, NEG)
    return loss[0, 0]


# speedup vs baseline: 1.0332x; 1.0187x over previous
import functools

import jax
import jax.numpy as jnp
from jax import lax
from jax.experimental import pallas as pl
from jax.experimental.pallas import tpu as pltpu
from jax.experimental.pallas import tpu_sc as plsc

NC = 2
NS = 16
NW = NC * NS
LANES = 16


def _sc_ctx_pool(ctx_idx, emb_in, B, CTX, D):
    bpw = B // NW
    CB = 16
    nch = bpw // CB
    nseg = D // LANES
    inv_ctx = jnp.float32(1.0 / CTX)

    mesh = plsc.VectorSubcoreMesh(core_axis_name="c", subcore_axis_name="s")

    @functools.partial(
        pl.kernel,
        mesh=mesh,
        compiler_params=pltpu.CompilerParams(
            needs_layout_passes=False, use_tc_tiling_on_sc=False),
        out_type=jax.ShapeDtypeStruct((B, D), jnp.float32),
        scratch_types=[
            pltpu.VMEM((2, CB, CTX), jnp.int32),
            pltpu.VMEM((2, CB * CTX, D), jnp.float32),
            pltpu.VMEM((2, CB, D), jnp.float32),
            pltpu.SemaphoreType.DMA,
            pltpu.SemaphoreType.DMA,
            pltpu.SemaphoreType.DMA,
            pltpu.SemaphoreType.DMA,
            pltpu.SemaphoreType.DMA,
            pltpu.SemaphoreType.DMA,
        ],
    )
    def k_ctx(ctx_i_hbm, ein_hbm, out_hbm, ctxi_v, ctxr_v, ctxv_v,
              sem0, sem1, semi0, semi1, semo0, semo1):
        wid = lax.axis_index("s") * NC + lax.axis_index("c")
        sems = (sem0, sem1)
        semis = (semi0, semi1)
        semos = (semo0, semo1)

        def idx_load(c, p):
            b0 = wid * bpw + c * CB
            pltpu.async_copy(ctx_i_hbm.at[pl.ds(b0, CB)], ctxi_v.at[p],
                             semis[p])

        def idx_wait(p):
            pltpu.make_async_copy(ctx_i_hbm.at[pl.ds(wid * bpw, CB)],
                                  ctxi_v.at[p], semis[p]).wait()

        def fire(c, p):
            for b in range(CB):
                pltpu.async_copy(ein_hbm.at[ctxi_v.at[p, b]],
                                 ctxr_v.at[p, pl.ds(b * CTX, CTX)], sems[p])

        def drain(p):
            for b in range(CB):
                pltpu.make_async_copy(ein_hbm.at[ctxi_v.at[p, b]],
                                      ctxr_v.at[p, pl.ds(b * CTX, CTX)],
                                      sems[p]).wait()

        def out_start(c, p):
            pltpu.async_copy(ctxv_v.at[p],
                             out_hbm.at[pl.ds(wid * bpw + c * CB, CB)],
                             semos[p])

        def out_wait(p):
            pltpu.make_async_copy(ctxv_v.at[p],
                                  out_hbm.at[pl.ds(wid * bpw, CB)],
                                  semos[p]).wait()

        def compute(c, p):
            def bbody(b, carry):
                base = b * CTX
                for s in range(nseg):
                    v = [ctxr_v[p, base + j, pl.ds(s * LANES, LANES)]
                         for j in range(CTX)]
                    while len(v) > 1:
                        v = [v[i] + v[i + 1] for i in range(0, len(v) - 1, 2)] \
                            + ([v[-1]] if len(v) % 2 else [])
                    ctxv_v[p, b, pl.ds(s * LANES, LANES)] = v[0] * inv_ctx
                return carry

            lax.fori_loop(0, CB, bbody, 0, unroll=2)

        idx_load(0, 0)
        idx_wait(0)
        fire(0, 0)
        idx_load(1, 1)

        def pair(i, carry):
            for pp in range(2):
                c = i * 2 + pp
                drain(pp)

                @pl.when(c + 2 < nch)
                def _():
                    idx_load(c + 2, pp)

                @pl.when(c + 1 < nch)
                def _():
                    idx_wait(1 - pp)
                    fire(c + 1, 1 - pp)

                @pl.when(c >= 2)
                def _():
                    out_wait(pp)

                compute(c, pp)
                out_start(c, pp)
            return carry

        lax.fori_loop(0, nch // 2, pair, 0)
        out_wait(0)
        out_wait(1)

    return k_ctx(ctx_idx, emb_in)


def _sc_score2(tgt_idx, neg_idx, ctxv, emb_out, B, NEG, D):
    bpw = B // NW
    CB = 16
    nch = bpw // CB
    nseg = D // LANES

    mesh = plsc.VectorSubcoreMesh(core_axis_name="c", subcore_axis_name="s")

    @functools.partial(
        pl.kernel,
        mesh=mesh,
        compiler_params=pltpu.CompilerParams(
            needs_layout_passes=False, use_tc_tiling_on_sc=False),
        out_type=jax.ShapeDtypeStruct((B, 32), jnp.float32),
        scratch_types=[
            pltpu.VMEM((2, CB, NEG), jnp.int32),
            pltpu.VMEM((2, CB), jnp.int32),
            pltpu.VMEM((2, CB * NEG, D), jnp.float32),
            pltpu.VMEM((2, CB, D), jnp.float32),
            pltpu.VMEM((2, CB, D), jnp.float32),
            pltpu.VMEM((2, CB, 32), jnp.float32),
            pltpu.SemaphoreType.DMA,
            pltpu.SemaphoreType.DMA,
            pltpu.SemaphoreType.DMA,
            pltpu.SemaphoreType.DMA,
            pltpu.SemaphoreType.DMA,
            pltpu.SemaphoreType.DMA,
        ],
    )
    def k_score(tgt_i_hbm, neg_i_hbm, ctxv_hbm, eout_hbm, out_hbm,
                negi_v, tgti_v, negr_v, tgtr_v, ctxvb_v, sc_v,
                sem0, sem1, semi0, semi1, semo0, semo1):
        wid = lax.axis_index("s") * NC + lax.axis_index("c")
        sems = (sem0, sem1)
        semis = (semi0, semi1)
        semos = (semo0, semo1)

        def idx_load(c, p):
            b0 = wid * bpw + c * CB
            pltpu.async_copy(neg_i_hbm.at[pl.ds(b0, CB)], negi_v.at[p],
                             semis[p])
            pltpu.async_copy(tgt_i_hbm.at[pl.ds(b0, CB)], tgti_v.at[p],
                             semis[p])

        def idx_wait(p):
            b0 = wid * bpw
            pltpu.make_async_copy(neg_i_hbm.at[pl.ds(b0, CB)], negi_v.at[p],
                                  semis[p]).wait()
            pltpu.make_async_copy(tgt_i_hbm.at[pl.ds(b0, CB)], tgti_v.at[p],
                                  semis[p]).wait()

        def fire(c, p):
            b0 = wid * bpw + c * CB
            for b in range(CB):
                pltpu.async_copy(eout_hbm.at[negi_v.at[p, b]],
                                 negr_v.at[p, pl.ds(b * NEG, NEG)], sems[p])
            pltpu.async_copy(eout_hbm.at[tgti_v.at[p]], tgtr_v.at[p], sems[p])
            pltpu.async_copy(ctxv_hbm.at[pl.ds(b0, CB)], ctxvb_v.at[p],
                             sems[p])

        def drain(p):
            b0 = wid * bpw
            for b in range(CB):
                pltpu.make_async_copy(eout_hbm.at[negi_v.at[p, b]],
                                      negr_v.at[p, pl.ds(b * NEG, NEG)],
                                      sems[p]).wait()
            pltpu.make_async_copy(eout_hbm.at[tgti_v.at[p]], tgtr_v.at[p],
                                  sems[p]).wait()
            pltpu.make_async_copy(ctxv_hbm.at[pl.ds(b0, CB)], ctxvb_v.at[p],
                                  sems[p]).wait()

        def score_out(c, p):
            pltpu.async_copy(sc_v.at[p],
                             out_hbm.at[pl.ds(wid * bpw + c * CB, CB)],
                             semos[p])

        def score_wait(p):
            pltpu.make_async_copy(sc_v.at[p],
                                  out_hbm.at[pl.ds(wid * bpw, CB)],
                                  semos[p]).wait()

        lane = lax.iota(jnp.int32, LANES)
        lane_neg = lane * NEG
        zero = jnp.zeros((LANES,), jnp.float32)

        def compute(c, p):
            def dbody(d, carry):
                pos = carry[0]
                negs = carry[1:]
                dcol = jnp.broadcast_to(d, (LANES,))
                acc = plsc.load_gather(ctxvb_v.at[p], [lane, dcol])
                tcol = plsc.load_gather(tgtr_v.at[p], [lane, dcol])
                gs = [plsc.load_gather(negr_v.at[p], [lane_neg + k, dcol])
                      for k in range(NEG)]
                pos = pos + acc * tcol
                negs = [n + acc * g for n, g in zip(negs, gs)]
                return [pos] + negs

            res = lax.fori_loop(0, D, dbody, [zero] * (NEG + 1), unroll=2)
            plsc.store_scatter(sc_v.at[p],
                               [lane, jnp.broadcast_to(0, (LANES,))], -res[0])
            for k in range(NEG):
                plsc.store_scatter(sc_v.at[p],
                                   [lane, jnp.broadcast_to(k + 1, (LANES,))],
                                   res[k + 1])

        for q in range(2):
            for z in range(CB):
                sc_v[q, z, pl.ds(16, 16)] = jnp.zeros((LANES,), jnp.float32)

        idx_load(0, 0)
        idx_wait(0)
        fire(0, 0)
        idx_load(1, 1)

        def pair(i, carry):
            for pp in range(2):
                c = i * 2 + pp
                drain(pp)

                @pl.when(c + 2 < nch)
                def _():
                    idx_load(c + 2, pp)

                @pl.when(c + 1 < nch)
                def _():
                    idx_wait(1 - pp)
                    fire(c + 1, 1 - pp)

                @pl.when(c >= 2)
                def _():
                    score_wait(pp)

                compute(c, pp)
                score_out(c, pp)
            return carry

        lax.fori_loop(0, nch // 2, pair, 0)
        score_wait(0)
        score_wait(1)

    return k_score(tgt_idx, neg_idx, ctxv, emb_out)


def _tc_loss(scores, B, NEG):
    RB = 2048
    grid = B // RB

    def body(s_ref, o_ref):
        i = pl.program_id(0)
        x = s_ref[...]
        col = lax.broadcasted_iota(jnp.int32, x.shape, 1)
        sp = jnp.maximum(x, 0.0) + jnp.log1p(jnp.exp(-jnp.abs(x)))
        sp = jnp.where(col < NEG + 1, sp, 0.0)
        part = jnp.sum(sp)

        @pl.when(i == 0)
        def _():
            o_ref[0, 0] = 0.0

        o_ref[0, 0] += part

        @pl.when(i == grid - 1)
        def _():
            o_ref[0, 0] = o_ref[0, 0] * jnp.float32(1.0 / B)

    return pl.pallas_call(
        body,
        grid=(grid,),
        in_specs=[pl.BlockSpec((RB, 32), lambda i: (i, 0))],
        out_specs=pl.BlockSpec(memory_space=pltpu.SMEM),
        out_shape=jax.ShapeDtypeStruct((1, 1), jnp.float32),
    )(scores)


def kernel(context_words, target_words, negative_samples, emb_in, emb_out):
    B, CTX = context_words.shape
    NEG = negative_samples.shape[1]
    D = emb_in.shape[1]

    ctx_idx = context_words.astype(jnp.int32)
    neg_idx = negative_samples.astype(jnp.int32)
    tgt_idx = target_words.astype(jnp.int32)

    ctxv = _sc_ctx_pool(ctx_idx, emb_in, B, CTX, D)
    scores = _sc_score2(tgt_idx, neg_idx, ctxv, emb_out, B, NEG, D)
    loss = _tc_loss(scores, B, NEG)
    return loss[0, 0]
